# baseline (speedup 1.0000x reference)
import functools

import jax
import jax.numpy as jnp
from jax import lax
from jax.experimental import pallas as pl
from jax.experimental.pallas import tpu as pltpu
from jax.experimental.pallas import tpu_sc as plsc

_LOG2_W = 5
_LOG2_MAX_W = 5

_BLK_B = 4
_SPLIT = 24


def _sc_gather_pe(pos_embed, P):
    _, D = pos_embed.shape
    info = plsc.get_sparse_core_info()
    NC, NS, L = info.num_cores, info.num_subcores, info.num_lanes
    NW = NC * NS
    rows_per_w = P // NW
    mesh = plsc.VectorSubcoreMesh(core_axis_name="c", subcore_axis_name="s")

    @functools.partial(
        pl.kernel,
        mesh=mesh,
        out_type=jax.ShapeDtypeStruct((P, D), jnp.float32),
        scratch_types=[
            pltpu.VMEM((rows_per_w,), jnp.int32),
            pltpu.VMEM((rows_per_w, D), jnp.float32),
            pltpu.SemaphoreType.DMA,
        ],
    )
    def gather_kernel(table_hbm, out_hbm, idx_v, rows_v, sem):
        wid = lax.axis_index("s") * NC + lax.axis_index("c")
        base = wid * rows_per_w
        for j in range(rows_per_w // L):
            p = base + j * L + lax.iota(jnp.int32, L)
            idx_v[pl.ds(j * L, L)] = ((p >> _LOG2_W) << _LOG2_MAX_W) + (
                p & ((1 << _LOG2_W) - 1)
            )
        pltpu.async_copy(table_hbm.at[idx_v], rows_v, sem).wait()
        pltpu.sync_copy(rows_v, out_hbm.at[pl.ds(base, rows_per_w)])

    return gather_kernel(pos_embed)


def _add_body(x_ref, pe_ref, o_ref):
    o_ref[...] = x_ref[...] + pe_ref[...]


def _add_tail_body(x_ref, pe_ref, prev_ref, o_ref):
    del prev_ref
    o_ref[...] = x_ref[...] + pe_ref[...]


def kernel(x, H, W, pos_embed):
    B, P, D = x.shape
    pe_sc = _sc_gather_pe(pos_embed, P)

    head = pl.pallas_call(
        _add_body,
        grid=(_SPLIT // _BLK_B,),
        in_specs=[
            pl.BlockSpec((_BLK_B, P, D), lambda i: (i, 0, 0)),
            pl.BlockSpec((P, D), lambda i: (0, 0)),
        ],
        out_specs=pl.BlockSpec((_BLK_B, P, D), lambda i: (i, 0, 0)),
        out_shape=jax.ShapeDtypeStruct((B, P, D), x.dtype),
    )(x, pos_embed[:P])

    off = _SPLIT // _BLK_B
    return pl.pallas_call(
        _add_tail_body,
        grid=((B - _SPLIT) // _BLK_B,),
        in_specs=[
            pl.BlockSpec((_BLK_B, P, D), lambda i: (i + off, 0, 0)),
            pl.BlockSpec((P, D), lambda i: (0, 0)),
            pl.BlockSpec(memory_space=pl.ANY),
        ],
        out_specs=pl.BlockSpec((_BLK_B, P, D), lambda i: (i + off, 0, 0)),
        out_shape=jax.ShapeDtypeStruct((B, P, D), x.dtype),
        input_output_aliases={2: 0},
    )(x, pe_sc, head)


# speedup vs baseline: 1.2957x; 1.2957x over previous
import jax
import jax.numpy as jnp
from jax.experimental import pallas as pl

_BLK_B = 8
_BLK_P = 512


def _add_body(x_ref, pe_ref, o_ref):
    o_ref[...] = x_ref[...] + pe_ref[...]


def kernel(x, H, W, pos_embed):
    B, P, D = x.shape
    return pl.pallas_call(
        _add_body,
        grid=(P // _BLK_P, B // _BLK_B),
        in_specs=[
            pl.BlockSpec((_BLK_B, _BLK_P, D), lambda p, b: (b, p, 0)),
            pl.BlockSpec((_BLK_P, D), lambda p, b: (p, 0)),
        ],
        out_specs=pl.BlockSpec((_BLK_B, _BLK_P, D), lambda p, b: (b, p, 0)),
        out_shape=jax.ShapeDtypeStruct((B, P, D), x.dtype),
    )(x, pos_embed[:P])


# speedup vs baseline: 1.3064x; 1.0083x over previous
import jax
import jax.numpy as jnp
from jax.experimental import pallas as pl

_BLK_B = 4


def _add_body(x_ref, pe_ref, o_ref):
    o_ref[...] = x_ref[...] + pe_ref[...]


def kernel(x, H, W, pos_embed):
    B, P, D = x.shape
    return pl.pallas_call(
        _add_body,
        grid=(B // _BLK_B,),
        in_specs=[
            pl.BlockSpec((_BLK_B, P, D), lambda i: (i, 0, 0)),
            pl.BlockSpec((P, D), lambda i: (0, 0)),
        ],
        out_specs=pl.BlockSpec((_BLK_B, P, D), lambda i: (i, 0, 0)),
        out_shape=jax.ShapeDtypeStruct((B, P, D), x.dtype),
    )(x, pos_embed[:P])
